# baseline (speedup 1.0000x reference)
import jax
import jax.numpy as jnp
from jax.experimental import pallas as pl
from jax.experimental.pallas import tpu as pltpu

_HID = 56
_TBL = 524288
_TBC = 4096
_QUAD = 4 * _TBC


def _round_up(n, m):
    return (n + m - 1) // m * m


def _mlp_lanes_kernel(xt_ref, w1q_ref, b1q_ref, w2q_ref, b2_ref, ot_ref):
    w1q = w1q_ref[...].astype(jnp.bfloat16)
    b1q = b1q_ref[...].astype(jnp.bfloat16)
    w2q = w2q_ref[...].astype(jnp.bfloat16)
    b2 = b2_ref[...]
    tbl = xt_ref.shape[1]
    for q in range(0, tbl, _QUAD):
        w = min(_TBC, tbl - q)
        los = [min(q + a * _TBC, tbl - w) for a in range(4)]
        xq = jnp.concatenate([xt_ref[:, lo:lo + w] for lo in los],
                             axis=0).astype(jnp.bfloat16)
        h = jnp.dot(w1q, xq, preferred_element_type=jnp.float32)
        hb = jnp.maximum(h.astype(jnp.bfloat16) + b1q, 0)
        y4 = jnp.dot(w2q, hb, preferred_element_type=jnp.float32)
        for a, lo in enumerate(los):
            ot_ref[:, lo:lo + w] = y4[2 * a:2 * a + 2, :] + b2


def kernel(x, w1p, b1p, w2p, b2p):
    w1c = w1p[:_HID]
    b1c = b1p[:_HID]
    w2c = w2p[:, :_HID]
    eye4 = jnp.eye(4, dtype=jnp.float32)
    w1q = jnp.kron(eye4, w1c)
    b1q = jnp.tile(b1c, (4, 1))
    w2q = jnp.kron(eye4, w2c)

    B = x.shape[0]
    xt = x.T
    b_pad = _round_up(B, 512)
    if b_pad != B:
        xt = jnp.pad(xt, ((0, 0), (0, b_pad - B)))
    if b_pad % _TBL == 0:
        tbl = _TBL
    else:
        tbl = b_pad

    yt = pl.pallas_call(
        _mlp_lanes_kernel,
        out_shape=jax.ShapeDtypeStruct((2, b_pad), jnp.float32),
        grid=(b_pad // tbl,),
        in_specs=[
            pl.BlockSpec((4, tbl), lambda i: (0, i)),
            pl.BlockSpec(w1q.shape, lambda i: (0, 0)),
            pl.BlockSpec(b1q.shape, lambda i: (0, 0)),
            pl.BlockSpec(w2q.shape, lambda i: (0, 0)),
            pl.BlockSpec(b2p.shape, lambda i: (0, 0)),
        ],
        out_specs=pl.BlockSpec((2, tbl), lambda i: (0, i)),
        compiler_params=pltpu.CompilerParams(
            dimension_semantics=("parallel",)),
    )(xt, w1q, b1q, w2q, b2p)

    if b_pad != B:
        yt = yt[:, :B]
    return yt.T


# speedup vs baseline: 1.0134x; 1.0134x over previous
import jax
import jax.numpy as jnp
from jax.experimental import pallas as pl
from jax.experimental.pallas import tpu as pltpu

_HID = 56
_TBL = 262144
_TBC = 4096
_QUAD = 4 * _TBC


def _round_up(n, m):
    return (n + m - 1) // m * m


def _mlp_lanes_kernel(xt_ref, w1q_ref, b1q_ref, w2q_ref, b2_ref, ot_ref):
    w1q = w1q_ref[...].astype(jnp.bfloat16)
    b1q = b1q_ref[...].astype(jnp.bfloat16)
    w2q = w2q_ref[...].astype(jnp.bfloat16)
    b2 = b2_ref[...]
    tbl = xt_ref.shape[1]
    for q in range(0, tbl, _QUAD):
        w = min(_TBC, tbl - q)
        los = [min(q + a * _TBC, tbl - w) for a in range(4)]
        xq = jnp.concatenate([xt_ref[:, lo:lo + w] for lo in los],
                             axis=0).astype(jnp.bfloat16)
        h = jnp.dot(w1q, xq, preferred_element_type=jnp.float32)
        hb = jnp.maximum(h.astype(jnp.bfloat16) + b1q, 0)
        y4 = jnp.dot(w2q, hb, preferred_element_type=jnp.float32)
        for a, lo in enumerate(los):
            ot_ref[:, lo:lo + w] = y4[2 * a:2 * a + 2, :] + b2


def kernel(x, w1p, b1p, w2p, b2p):
    w1c = w1p[:_HID]
    b1c = b1p[:_HID]
    w2c = w2p[:, :_HID]
    eye4 = jnp.eye(4, dtype=jnp.float32)
    w1q = jnp.kron(eye4, w1c)
    b1q = jnp.tile(b1c, (4, 1))
    w2q = jnp.kron(eye4, w2c)

    B = x.shape[0]
    xt = x.T
    b_pad = _round_up(B, 512)
    if b_pad != B:
        xt = jnp.pad(xt, ((0, 0), (0, b_pad - B)))
    if b_pad % _TBL == 0:
        tbl = _TBL
    else:
        tbl = b_pad

    yt = pl.pallas_call(
        _mlp_lanes_kernel,
        out_shape=jax.ShapeDtypeStruct((2, b_pad), jnp.float32),
        grid=(b_pad // tbl,),
        in_specs=[
            pl.BlockSpec((4, tbl), lambda i: (0, i)),
            pl.BlockSpec(w1q.shape, lambda i: (0, 0)),
            pl.BlockSpec(b1q.shape, lambda i: (0, 0)),
            pl.BlockSpec(w2q.shape, lambda i: (0, 0)),
            pl.BlockSpec(b2p.shape, lambda i: (0, 0)),
        ],
        out_specs=pl.BlockSpec((2, tbl), lambda i: (0, i)),
        compiler_params=pltpu.CompilerParams(
            dimension_semantics=("parallel",)),
    )(xt, w1q, b1q, w2q, b2p)

    if b_pad != B:
        yt = yt[:, :B]
    return yt.T
